# baseline (speedup 1.0000x reference)
import functools

import jax
import jax.numpy as jnp
from jax import lax
from jax.experimental import pallas as pl
from jax.experimental.pallas import tpu as pltpu
from jax.experimental.pallas import tpu_sc as plsc

N = 10000
E = 320000
D = 128

NC = 2
NS = 16
L = 16

EPT = E // NS
CE = 256
NCHP = (EPT + CE - 1) // CE
EPTP = NCHP * CE
ROW0_STEP = 624
RCH = 48


def _gelu(x):
    return 0.5 * x * (1.0 + lax.erf(x * 0.7071067811865476))


def _rsqrt_newton(v):
    i = plsc.bitcast(v, jnp.int32)
    i = 0x5F3759DF - lax.shift_right_logical(i, 1)
    y = plsc.bitcast(i, jnp.float32)
    for _ in range(4):
        y = y * (1.5 - 0.5 * v * y * y)
    return y


def _make_deg_kernel():
    mesh = plsc.VectorSubcoreMesh(core_axis_name="c", subcore_axis_name="s")

    scratch = dict(
        dstall=pltpu.VMEM((NCHP, CE), jnp.int32),
        hist=pltpu.VMEM((N + L,), jnp.float32),
        degbuf=pltpu.VMEM((640,), jnp.float32),
        tmp640=pltpu.VMEM((640,), jnp.float32),
        disbuf=pltpu.VMEM((640,), jnp.float32),
        hists_sp=pltpu.VMEM_SHARED((NS, N), jnp.float32),
    )

    def body(dst_hbm, dis_hbm, dstall, hist, degbuf, tmp640, disbuf,
             hists_sp):
        c = lax.axis_index("c")
        s = lax.axis_index("s")
        row0 = s * ROW0_STEP
        zf = jnp.zeros((L,), jnp.float32)
        pltpu.sync_copy(dst_hbm.at[s], dstall)

        def zhist(i, _):
            hist[pl.ds(i * L, L)] = zf
            return 0
        lax.fori_loop(0, (N + L) // L, zhist, 0)
        ones = jnp.full((L,), 1.0, jnp.float32)

        def dchunk_body(j, _):
            for u in range(CE // L):
                idx = dstall[j, pl.ds(u * L, L)]
                plsc.addupdate_scatter(hist, [idx], ones)
            return 0
        lax.fori_loop(0, NCHP, dchunk_body, 0)
        pltpu.sync_copy(hist.at[pl.ds(0, N)], hists_sp.at[s])
        plsc.subcore_barrier()

        def zdeg(g, _):
            degbuf[pl.ds(g * L, L)] = zf
            return 0
        lax.fori_loop(0, 40, zdeg, 0)
        for t in range(NS):
            pltpu.sync_copy(hists_sp.at[t, pl.ds(row0, 640)], tmp640)
            def addt(g, _):
                degbuf[pl.ds(g * L, L)] = (degbuf[pl.ds(g * L, L)]
                                           + tmp640[pl.ds(g * L, L)])
                return 0
            lax.fori_loop(0, 40, addt, 0)

        def mkdis(g, _):
            v = degbuf[pl.ds(g * L, L)]
            y = _rsqrt_newton(v)
            disbuf[pl.ds(g * L, L)] = jnp.where(v > 0.5, y, 0.0)
            return 0
        lax.fori_loop(0, 40, mkdis, 0)

        @pl.when(c == 0)
        def _():
            pltpu.sync_copy(disbuf.at[pl.ds(0, ROW0_STEP)],
                            dis_hbm.at[pl.ds(row0, ROW0_STEP)])
            @pl.when(s == NS - 1)
            def _():
                pltpu.sync_copy(disbuf.at[pl.ds(ROW0_STEP, 16)],
                                dis_hbm.at[pl.ds(N - 16, 16)])

    return pl.kernel(
        body,
        out_type=[jax.ShapeDtypeStruct((N,), jnp.float32)],
        mesh=mesh,
        scratch_types=list(scratch.values()),
        compiler_params=pltpu.CompilerParams(
            use_tc_tiling_on_sc=False, needs_layout_passes=False),
    )


def _make_tag_kernel(F, npass):
    Fc = F // NC
    Fcb = Fc // npass
    UF = Fcb // L

    mesh = plsc.VectorSubcoreMesh(core_axis_name="c", subcore_axis_name="s")

    out_type = [
        jax.ShapeDtypeStruct((N, F), jnp.float32),
        jax.ShapeDtypeStruct((NC * npass * N, Fcb), jnp.float32),
    ]

    scratch = dict(
        bufA=pltpu.VMEM((CE, Fcb), jnp.float32),
        bufB=pltpu.VMEM((CE, Fcb), jnp.float32),
        dstall=pltpu.VMEM((NCHP, CE), jnp.int32),
        idxmod=pltpu.VMEM((NCHP, CE), jnp.int32),
        bufQ=pltpu.VMEM((RCH, Fcb), jnp.float32),
        bufAcc=pltpu.VMEM((RCH, Fcb), jnp.float32),
        zero16=pltpu.VMEM((L, Fcb), jnp.float32),
        disbuf=pltpu.VMEM((640,), jnp.float32),
        acc_sp=pltpu.VMEM_SHARED((N + L, Fcb), jnp.float32),
        semA=pltpu.SemaphoreType.DMA,
        semB=pltpu.SemaphoreType.DMA,
    )

    def body(*refs):
        (q_hbm, src_hbm, dst_hbm, dis_in_hbm, out_hbm, s_hbm,
         bufA, bufB, dstall, idxmod, bufQ, bufAcc, zero16, disbuf,
         acc_sp, semA, semB) = refs

        c = lax.axis_index("c")
        s = lax.axis_index("s")
        row0 = s * ROW0_STEP
        zf = jnp.zeros((L,), jnp.float32)

        for r in range(L):
            for u in range(UF):
                zero16[r, pl.ds(u * L, L)] = zf
        pltpu.sync_copy(dst_hbm.at[s], dstall)
        pltpu.sync_copy(dis_in_hbm.at[pl.ds(row0, 640)], disbuf)

        def combine_rows(r0_local, nrows, k, is_first, p):
            r0 = row0 + r0_local
            colbase = c * Fc + p * Fcb
            blk = c * npass + p
            bq_dma = bufQ if nrows == RCH else bufQ.at[pl.ds(0, nrows)]
            ba_dma = bufAcc if nrows == RCH else bufAcc.at[pl.ds(0, nrows)]
            pltpu.sync_copy(
                q_hbm.at[k, pl.ds(r0, nrows), pl.ds(colbase, Fcb)], bq_dma)
            if not is_first:
                pltpu.sync_copy(acc_sp.at[pl.ds(r0, nrows)], ba_dma)
                for z in range(nrows // L):
                    pltpu.sync_copy(zero16, acc_sp.at[pl.ds(r0 + z * L, L)])

            def rowfn(g, _):
                dvec = disbuf[pl.ds(r0_local + g * L, L)]
                for r16 in range(L):
                    r = g * L + r16
                    dv = dvec[r16]
                    for u in range(UF):
                        sl = pl.ds(u * L, L)
                        t = (bufQ[r, sl] if is_first
                             else bufQ[r, sl] + dv * bufAcc[r, sl])
                        if k > 0:
                            t = dv * t
                        bufQ[r, sl] = t
                return 0
            lax.fori_loop(0, nrows // L, rowfn, 0)

            if k > 0:
                pltpu.sync_copy(bq_dma, s_hbm.at[pl.ds(blk * N + r0, nrows)])
            else:
                pltpu.sync_copy(
                    bq_dma, out_hbm.at[pl.ds(r0, nrows), pl.ds(colbase, Fcb)])

        NRC = ROW0_STEP // RCH

        def combine_phase(k, is_first, p):
            def ch_body(ch, _):
                combine_rows(ch * RCH, RCH, k, is_first, p)
                return 0
            lax.fori_loop(0, NRC, ch_body, 0)
            @pl.when(s == NS - 1)
            def _():
                combine_rows(NRC * RCH, L, k, is_first, p)

        def gather_scatter_phase(p):
            cN = (c * npass + p) * N

            GB = (bufA, bufB)
            GS = (semA, semB)

            def fire_g(j, buf, sem):
                pltpu.async_copy(s_hbm.at[idxmod.at[j]], buf, sem)

            def drain(buf, sem):
                pltpu.make_async_copy(s_hbm.at[pl.ds(0, CE)], buf, sem).wait()

            def scat(j, buf):
                pltpu.sync_copy(buf, acc_sp.at[dstall.at[j]], add=True)

            pltpu.sync_copy(src_hbm.at[s], idxmod)

            def mkidx(j, _):
                for u in range(CE // L):
                    sl = pl.ds(u * L, L)
                    idxmod[j, sl] = idxmod[j, sl] + cN
                return 0
            lax.fori_loop(0, NCHP, mkidx, 0)

            fire_g(0, GB[0], GS[0])

            def pair(i, _):
                fire_g(2 * i + 1, GB[1], GS[1])
                drain(GB[0], GS[0])
                scat(2 * i, GB[0])
                fire_g(2 * i + 2, GB[0], GS[0])
                drain(GB[1], GS[1])
                scat(2 * i + 1, GB[1])
                return 0
            lax.fori_loop(0, NCHP // 2, pair, 0)
            drain(GB[0], GS[0])
            scat(NCHP - 1, GB[0])

        def init_zero_acc(ch, _):
            r0 = row0 + ch * RCH
            for z in range(RCH // L):
                pltpu.sync_copy(zero16, acc_sp.at[pl.ds(r0 + z * L, L)])
            return 0
        lax.fori_loop(0, ROW0_STEP // RCH, init_zero_acc, 0)
        @pl.when(s == NS - 1)
        def _():
            pltpu.sync_copy(zero16, acc_sp.at[pl.ds(row0 + ROW0_STEP, L)])
        for p in range(npass):
            combine_phase(3, is_first=True, p=p)
        plsc.subcore_barrier()

        for k in (2, 1, 0):
            for p in range(npass):
                gather_scatter_phase(p)
                plsc.subcore_barrier()
                combine_phase(k, is_first=False, p=p)
                plsc.subcore_barrier()

    kfn = pl.kernel(
        body,
        out_type=out_type,
        mesh=mesh,
        scratch_types=list(scratch.values()),
        compiler_params=pltpu.CompilerParams(
            use_tc_tiling_on_sc=False, needs_layout_passes=False,
            internal_scratch_in_bytes=256 * 1024),
    )

    return kfn


_deg = _make_deg_kernel()
_tag1 = _make_tag_kernel(256, npass=2)
_tag2 = _make_tag_kernel(64, npass=1)


def _stats_body(x_ref, o_ref):
    xm = jnp.mean(x_ref[...], axis=0, keepdims=True)
    xv = jnp.mean(x_ref[...] * x_ref[...], axis=0, keepdims=True) - xm * xm
    o_ref[0:1, :] = xm
    o_ref[1:2, :] = xv


def _stats(x):
    return pl.pallas_call(
        _stats_body,
        out_shape=jax.ShapeDtypeStruct((2, D), jnp.float32),
    )(x)


NB = 1000


def _prologue_body(x_ref, st_ref, g_ref, be_ref, w1_ref, b1_ref, cw_ref,
                   cb_ref, q_ref, h1_ref):
    j = pl.program_id(1)

    @pl.when(j == 0)
    def _():
        mean = st_ref[0:1, :]
        var = st_ref[1:2, :]
        inv = g_ref[...] * lax.rsqrt(var + 1e-5)
        xn = (x_ref[...] - mean) * inv + be_ref[...]
        h1_ref[...] = _gelu(
            jnp.dot(xn, w1_ref[...], preferred_element_type=jnp.float32)
            + b1_ref[...])

    q = jnp.dot(h1_ref[...], cw_ref[0], preferred_element_type=jnp.float32)

    @pl.when(j == 0)
    def _():
        q_ref[0] = q + cb_ref[...]

    @pl.when(j > 0)
    def _():
        q_ref[0] = q


def _prologue(x, stats, gamma, beta, W1, b1, cW1, cb1):
    return pl.pallas_call(
        _prologue_body,
        grid=(N // NB, 4),
        in_specs=[
            pl.BlockSpec((NB, D), lambda i, j: (i, 0)),
            pl.BlockSpec((2, D), lambda i, j: (0, 0)),
            pl.BlockSpec((1, D), lambda i, j: (0, 0)),
            pl.BlockSpec((1, D), lambda i, j: (0, 0)),
            pl.BlockSpec((D, 512), lambda i, j: (0, 0)),
            pl.BlockSpec((1, 512), lambda i, j: (0, 0)),
            pl.BlockSpec((1, 512, 256), lambda i, j: (j, 0, 0)),
            pl.BlockSpec((1, 256), lambda i, j: (0, 0)),
        ],
        out_specs=pl.BlockSpec((1, NB, 256), lambda i, j: (j, i, 0)),
        out_shape=jax.ShapeDtypeStruct((4, N, 256), jnp.float32),
        scratch_shapes=[pltpu.VMEM((NB, 512), jnp.float32)],
    )(x, stats, gamma.reshape(1, D), beta.reshape(1, D), W1,
      b1.reshape(1, 512), cW1, cb1.reshape(1, 256))


def _mid_body(t_ref, w2_ref, b2_ref, cw_ref, cb_ref, q_ref, h2_ref):
    j = pl.program_id(1)

    @pl.when(j == 0)
    def _():
        a = _gelu(t_ref[...])
        h2_ref[...] = _gelu(
            jnp.dot(a, w2_ref[...], preferred_element_type=jnp.float32)
            + b2_ref[...])

    q = jnp.dot(h2_ref[...], cw_ref[0], preferred_element_type=jnp.float32)

    @pl.when(j == 0)
    def _():
        q_ref[0] = q + cb_ref[...]

    @pl.when(j > 0)
    def _():
        q_ref[0] = q


def _mid(t1, W2, b2, cW2, cb2):
    return pl.pallas_call(
        _mid_body,
        grid=(N // NB, 4),
        in_specs=[
            pl.BlockSpec((NB, 256), lambda i, j: (i, 0)),
            pl.BlockSpec((256, 128), lambda i, j: (0, 0)),
            pl.BlockSpec((1, 128), lambda i, j: (0, 0)),
            pl.BlockSpec((1, 128, 64), lambda i, j: (j, 0, 0)),
            pl.BlockSpec((1, 64), lambda i, j: (0, 0)),
        ],
        out_specs=pl.BlockSpec((1, NB, 64), lambda i, j: (j, i, 0)),
        out_shape=jax.ShapeDtypeStruct((4, N, 64), jnp.float32),
        scratch_shapes=[pltpu.VMEM((NB, 128), jnp.float32)],
    )(t1, W2, b2.reshape(1, 128), cW2, cb2.reshape(1, 64))


def _epi_body(t_ref, x_ref, w3a_ref, w3b_ref, b3_ref, w4_ref, b4_ref, o_ref):
    h3 = _gelu(t_ref[...])
    e = _gelu(
        jnp.dot(h3, w3a_ref[...], preferred_element_type=jnp.float32)
        + jnp.dot(x_ref[...], w3b_ref[...], preferred_element_type=jnp.float32)
        + b3_ref[...])
    o_ref[...] = (jnp.dot(e, w4_ref[...], preferred_element_type=jnp.float32)
                  + b4_ref[...])


def _epilogue(t2, x, W3, b3, W4, b4):
    return pl.pallas_call(
        _epi_body,
        grid=(N // NB,),
        in_specs=[
            pl.BlockSpec((NB, 64), lambda i: (i, 0)),
            pl.BlockSpec((NB, D), lambda i: (i, 0)),
            pl.BlockSpec((64, 96), lambda i: (0, 0)),
            pl.BlockSpec((D, 96), lambda i: (0, 0)),
            pl.BlockSpec((1, 96), lambda i: (0, 0)),
            pl.BlockSpec((96, 48), lambda i: (0, 0)),
            pl.BlockSpec((1, 48), lambda i: (0, 0)),
        ],
        out_specs=pl.BlockSpec((NB, 48), lambda i: (i, 0)),
        out_shape=jax.ShapeDtypeStruct((N, 48), jnp.float32),
    )(t2, x, W3[:64], W3[64:], b3.reshape(1, 96), W4, b4.reshape(1, 48))


def kernel(x, edge_index, bn_gamma, bn_beta, W1, b1, cW1, cb1, W2, b2, cW2,
           cb2, W3, b3, W4, b4):
    pad = EPTP - EPT
    src = jnp.pad(edge_index[0].reshape(NS, EPT), ((0, 0), (0, pad)),
                  constant_values=0).reshape(NS, NCHP, CE)
    dst = jnp.pad(edge_index[1].reshape(NS, EPT), ((0, 0), (0, pad)),
                  constant_values=N).reshape(NS, NCHP, CE)

    (dis,) = _deg(dst)
    stats = _stats(x)
    q1 = _prologue(x, stats, bn_gamma, bn_beta, W1, b1, cW1, cb1)
    t1, _s1 = _tag1(q1, src, dst, dis)
    q2 = _mid(t1, W2, b2, cW2, cb2)
    t2, _s2 = _tag2(q2, src, dst, dis)
    return _epilogue(t2, x, W3, b3, W4, b4)


# speedup vs baseline: 1.0926x; 1.0926x over previous
import functools

import jax
import jax.numpy as jnp
from jax import lax
from jax.experimental import pallas as pl
from jax.experimental.pallas import tpu as pltpu
from jax.experimental.pallas import tpu_sc as plsc

N = 10000
E = 320000
D = 128

NC = 2
NS = 16
L = 16

EPT = E // NS
CE = 128
NCHP = (EPT + CE - 1) // CE
EPTP = NCHP * CE
ROW0_STEP = 640
RCH = 80
NPAD = 10240


def _gelu(x):
    return 0.5 * x * (1.0 + lax.erf(x * 0.7071067811865476))


def _rsqrt_newton(v):
    i = plsc.bitcast(v, jnp.int32)
    i = 0x5F3759DF - lax.shift_right_logical(i, 1)
    y = plsc.bitcast(i, jnp.float32)
    for _ in range(4):
        y = y * (1.5 - 0.5 * v * y * y)
    return y


def _make_deg_kernel():
    mesh = plsc.VectorSubcoreMesh(core_axis_name="c", subcore_axis_name="s")

    scratch = dict(
        dstall=pltpu.VMEM((NCHP, CE), jnp.int32),
        hist=pltpu.VMEM((N + L,), jnp.float32),
        degbuf=pltpu.VMEM((640,), jnp.float32),
        tmp640=pltpu.VMEM((640,), jnp.float32),
        disbuf=pltpu.VMEM((640,), jnp.float32),
        hists_sp=pltpu.VMEM_SHARED((NS, NPAD), jnp.float32),
    )

    def body(dst_hbm, dis_hbm, dstall, hist, degbuf, tmp640, disbuf,
             hists_sp):
        c = lax.axis_index("c")
        s = lax.axis_index("s")
        row0 = s * ROW0_STEP
        zf = jnp.zeros((L,), jnp.float32)
        pltpu.sync_copy(dst_hbm.at[s], dstall)

        def zhist(i, _):
            hist[pl.ds(i * L, L)] = zf
            return 0
        lax.fori_loop(0, (N + L) // L, zhist, 0)
        ones = jnp.full((L,), 1.0, jnp.float32)

        def dchunk_body(j, _):
            for u in range(CE // L):
                idx = dstall[j, pl.ds(u * L, L)]
                plsc.addupdate_scatter(hist, [idx], ones)
            return 0
        lax.fori_loop(0, NCHP, dchunk_body, 0)
        pltpu.sync_copy(hist.at[pl.ds(0, N)], hists_sp.at[s, pl.ds(0, N)])
        def zpad(g, _):
            hist[pl.ds(g * L, L)] = zf
            return 0
        lax.fori_loop(0, (NPAD - N) // L, zpad, 0)
        pltpu.sync_copy(hist.at[pl.ds(0, NPAD - N)],
                        hists_sp.at[s, pl.ds(N, NPAD - N)])
        plsc.subcore_barrier()

        def zdeg(g, _):
            degbuf[pl.ds(g * L, L)] = zf
            return 0
        lax.fori_loop(0, 40, zdeg, 0)
        for t in range(NS):
            pltpu.sync_copy(hists_sp.at[t, pl.ds(row0, 640)], tmp640)
            def addt(g, _):
                degbuf[pl.ds(g * L, L)] = (degbuf[pl.ds(g * L, L)]
                                           + tmp640[pl.ds(g * L, L)])
                return 0
            lax.fori_loop(0, 40, addt, 0)

        def mkdis(g, _):
            v = degbuf[pl.ds(g * L, L)]
            y = _rsqrt_newton(v)
            disbuf[pl.ds(g * L, L)] = jnp.where(v > 0.5, y, 0.0)
            return 0
        lax.fori_loop(0, 40, mkdis, 0)

        @pl.when(c == 0)
        def _():
            pltpu.sync_copy(disbuf, dis_hbm.at[pl.ds(row0, 640)])

    return pl.kernel(
        body,
        out_type=[jax.ShapeDtypeStruct((NPAD,), jnp.float32)],
        mesh=mesh,
        scratch_types=list(scratch.values()),
        compiler_params=pltpu.CompilerParams(
            use_tc_tiling_on_sc=False, needs_layout_passes=False),
    )


def _make_tag_kernel(F, npass):
    Fc = F // NC
    Fcb = Fc // npass
    UF = Fcb // L

    mesh = plsc.VectorSubcoreMesh(core_axis_name="c", subcore_axis_name="s")

    out_type = [
        jax.ShapeDtypeStruct((N, F), jnp.float32),
        jax.ShapeDtypeStruct((NC * npass * N, Fcb), jnp.float32),
    ]

    scratch = dict(
        bufA=pltpu.VMEM((CE, Fcb), jnp.float32),
        bufB=pltpu.VMEM((CE, Fcb), jnp.float32),
        dstall=pltpu.VMEM((NCHP, CE), jnp.int32),
        idxmod=pltpu.VMEM((NCHP, CE), jnp.int32),
        bufQ=pltpu.VMEM((RCH, Fcb), jnp.float32),
        bufAcc=pltpu.VMEM((RCH, Fcb), jnp.float32),
        bufQ2=pltpu.VMEM((RCH, Fcb), jnp.float32),
        bufAcc2=pltpu.VMEM((RCH, Fcb), jnp.float32),
        zero16=pltpu.VMEM((L, Fcb), jnp.float32),
        disbuf=pltpu.VMEM((640,), jnp.float32),
        acc_sp=pltpu.VMEM_SHARED((N + L, Fcb), jnp.float32),
        semA=pltpu.SemaphoreType.DMA,
        semB=pltpu.SemaphoreType.DMA,
    )

    def body(*refs):
        (q_hbm, src_hbm, dst_hbm, dis_in_hbm, out_hbm, s_hbm,
         bufA, bufB, dstall, idxmod, bufQ, bufAcc, bufQ2, bufAcc2,
         zero16, disbuf, acc_sp, semA, semB) = refs

        c = lax.axis_index("c")
        s = lax.axis_index("s")
        row0 = s * ROW0_STEP
        nrc = jnp.where(s == NS - 1, (N - 15 * ROW0_STEP) // RCH,
                        ROW0_STEP // RCH)
        zf = jnp.zeros((L,), jnp.float32)

        for r in range(L):
            for u in range(UF):
                zero16[r, pl.ds(u * L, L)] = zf
        pltpu.sync_copy(dst_hbm.at[s], dstall)
        pltpu.sync_copy(dis_in_hbm.at[pl.ds(row0, 640)], disbuf)

        def crow_compute(r0_local, nrows, k, is_first, bq, ba):
            def rowfn(g, _):
                dvec = disbuf[pl.ds(r0_local + g * L, L)]
                for r16 in range(L):
                    r = g * L + r16
                    dv = dvec[r16]
                    for u in range(UF):
                        sl = pl.ds(u * L, L)
                        t = (bq[r, sl] if is_first
                             else bq[r, sl] + dv * ba[r, sl])
                        if k > 0:
                            t = dv * t
                        bq[r, sl] = t
                return 0
            lax.fori_loop(0, nrows // L, rowfn, 0)

        def crow_writeback(r0_local, nrows, k, p, bq_dma):
            r0 = row0 + r0_local
            colbase = c * Fc + p * Fcb
            blk = c * npass + p
            if k > 0:
                pltpu.sync_copy(bq_dma, s_hbm.at[pl.ds(blk * N + r0, nrows)])
            else:
                pltpu.sync_copy(
                    bq_dma, out_hbm.at[pl.ds(r0, nrows), pl.ds(colbase, Fcb)])

        def combine_phase(k, is_first, p):
            colbase = c * Fc + p * Fcb

            def ch_body(ch, _):
                r0 = row0 + ch * RCH
                pltpu.sync_copy(
                    q_hbm.at[k, pl.ds(r0, RCH), pl.ds(colbase, Fcb)], bufQ)
                if not is_first:
                    pltpu.sync_copy(acc_sp.at[pl.ds(r0, RCH)], bufAcc)
                    for z in range(RCH // L):
                        pltpu.sync_copy(zero16,
                                        acc_sp.at[pl.ds(r0 + z * L, L)])
                crow_compute(ch * RCH, RCH, k, is_first, bufQ, bufAcc)
                crow_writeback(ch * RCH, RCH, k, p, bufQ)
                return 0
            lax.fori_loop(0, nrc, ch_body, 0)

        def gather_scatter_phase(p):
            cN = (c * npass + p) * N

            GB = (bufA, bufB)
            GS = (semA, semB)

            def fire_g(j, buf, sem):
                pltpu.async_copy(s_hbm.at[idxmod.at[j]], buf, sem)

            def drain(buf, sem):
                pltpu.make_async_copy(s_hbm.at[pl.ds(0, CE)], buf, sem).wait()

            def scat(j, buf):
                pltpu.sync_copy(buf, acc_sp.at[dstall.at[j]], add=True)

            pltpu.sync_copy(src_hbm.at[s], idxmod)

            def mkidx(j, _):
                for u in range(CE // L):
                    sl = pl.ds(u * L, L)
                    idxmod[j, sl] = idxmod[j, sl] + cN
                return 0
            lax.fori_loop(0, NCHP, mkidx, 0)

            fire_g(0, GB[0], GS[0])

            def pair(i, _):
                fire_g(2 * i + 1, GB[1], GS[1])
                drain(GB[0], GS[0])
                scat(2 * i, GB[0])
                fire_g(2 * i + 2, GB[0], GS[0])
                drain(GB[1], GS[1])
                scat(2 * i + 1, GB[1])
                return 0
            lax.fori_loop(0, NCHP // 2, pair, 0)
            drain(GB[0], GS[0])
            scat(NCHP - 1, GB[0])

        def init_zero_acc(ch, _):
            r0 = row0 + ch * RCH
            for z in range(RCH // L):
                pltpu.sync_copy(zero16, acc_sp.at[pl.ds(r0 + z * L, L)])
            return 0
        lax.fori_loop(0, nrc, init_zero_acc, 0)
        for p in range(npass):
            combine_phase(3, is_first=True, p=p)
        plsc.subcore_barrier()

        for k in (2, 1, 0):
            for p in range(npass):
                gather_scatter_phase(p)
                plsc.subcore_barrier()
                combine_phase(k, is_first=False, p=p)
                plsc.subcore_barrier()

    kfn = pl.kernel(
        body,
        out_type=out_type,
        mesh=mesh,
        scratch_types=list(scratch.values()),
        compiler_params=pltpu.CompilerParams(
            use_tc_tiling_on_sc=False, needs_layout_passes=False,
            internal_scratch_in_bytes=256 * 1024),
    )

    return kfn


_deg = _make_deg_kernel()
_tag1 = _make_tag_kernel(256, npass=2)
_tag2 = _make_tag_kernel(64, npass=1)


def _stats_body(x_ref, o_ref):
    xm = jnp.mean(x_ref[...], axis=0, keepdims=True)
    xv = jnp.mean(x_ref[...] * x_ref[...], axis=0, keepdims=True) - xm * xm
    o_ref[0:1, :] = xm
    o_ref[1:2, :] = xv


def _stats(x):
    return pl.pallas_call(
        _stats_body,
        out_shape=jax.ShapeDtypeStruct((2, D), jnp.float32),
    )(x)


NB = 1000


def _prologue_body(x_ref, st_ref, g_ref, be_ref, w1_ref, b1_ref, cw_ref,
                   cb_ref, q_ref, h1_ref):
    j = pl.program_id(1)

    @pl.when(j == 0)
    def _():
        mean = st_ref[0:1, :]
        var = st_ref[1:2, :]
        inv = g_ref[...] * lax.rsqrt(var + 1e-5)
        xn = (x_ref[...] - mean) * inv + be_ref[...]
        h1_ref[...] = _gelu(
            jnp.dot(xn, w1_ref[...], preferred_element_type=jnp.float32)
            + b1_ref[...])

    q = jnp.dot(h1_ref[...], cw_ref[0], preferred_element_type=jnp.float32)

    @pl.when(j == 0)
    def _():
        q_ref[0] = q + cb_ref[...]

    @pl.when(j > 0)
    def _():
        q_ref[0] = q


def _prologue(x, stats, gamma, beta, W1, b1, cW1, cb1):
    return pl.pallas_call(
        _prologue_body,
        grid=(N // NB, 4),
        in_specs=[
            pl.BlockSpec((NB, D), lambda i, j: (i, 0)),
            pl.BlockSpec((2, D), lambda i, j: (0, 0)),
            pl.BlockSpec((1, D), lambda i, j: (0, 0)),
            pl.BlockSpec((1, D), lambda i, j: (0, 0)),
            pl.BlockSpec((D, 512), lambda i, j: (0, 0)),
            pl.BlockSpec((1, 512), lambda i, j: (0, 0)),
            pl.BlockSpec((1, 512, 256), lambda i, j: (j, 0, 0)),
            pl.BlockSpec((1, 256), lambda i, j: (0, 0)),
        ],
        out_specs=pl.BlockSpec((1, NB, 256), lambda i, j: (j, i, 0)),
        out_shape=jax.ShapeDtypeStruct((4, N, 256), jnp.float32),
        scratch_shapes=[pltpu.VMEM((NB, 512), jnp.float32)],
    )(x, stats, gamma.reshape(1, D), beta.reshape(1, D), W1,
      b1.reshape(1, 512), cW1, cb1.reshape(1, 256))


def _mid_body(t_ref, w2_ref, b2_ref, cw_ref, cb_ref, q_ref, h2_ref):
    j = pl.program_id(1)

    @pl.when(j == 0)
    def _():
        a = _gelu(t_ref[...])
        h2_ref[...] = _gelu(
            jnp.dot(a, w2_ref[...], preferred_element_type=jnp.float32)
            + b2_ref[...])

    q = jnp.dot(h2_ref[...], cw_ref[0], preferred_element_type=jnp.float32)

    @pl.when(j == 0)
    def _():
        q_ref[0] = q + cb_ref[...]

    @pl.when(j > 0)
    def _():
        q_ref[0] = q


def _mid(t1, W2, b2, cW2, cb2):
    return pl.pallas_call(
        _mid_body,
        grid=(N // NB, 4),
        in_specs=[
            pl.BlockSpec((NB, 256), lambda i, j: (i, 0)),
            pl.BlockSpec((256, 128), lambda i, j: (0, 0)),
            pl.BlockSpec((1, 128), lambda i, j: (0, 0)),
            pl.BlockSpec((1, 128, 64), lambda i, j: (j, 0, 0)),
            pl.BlockSpec((1, 64), lambda i, j: (0, 0)),
        ],
        out_specs=pl.BlockSpec((1, NB, 64), lambda i, j: (j, i, 0)),
        out_shape=jax.ShapeDtypeStruct((4, N, 64), jnp.float32),
        scratch_shapes=[pltpu.VMEM((NB, 128), jnp.float32)],
    )(t1, W2, b2.reshape(1, 128), cW2, cb2.reshape(1, 64))


def _epi_body(t_ref, x_ref, w3a_ref, w3b_ref, b3_ref, w4_ref, b4_ref, o_ref):
    h3 = _gelu(t_ref[...])
    e = _gelu(
        jnp.dot(h3, w3a_ref[...], preferred_element_type=jnp.float32)
        + jnp.dot(x_ref[...], w3b_ref[...], preferred_element_type=jnp.float32)
        + b3_ref[...])
    o_ref[...] = (jnp.dot(e, w4_ref[...], preferred_element_type=jnp.float32)
                  + b4_ref[...])


def _epilogue(t2, x, W3, b3, W4, b4):
    return pl.pallas_call(
        _epi_body,
        grid=(N // NB,),
        in_specs=[
            pl.BlockSpec((NB, 64), lambda i: (i, 0)),
            pl.BlockSpec((NB, D), lambda i: (i, 0)),
            pl.BlockSpec((64, 96), lambda i: (0, 0)),
            pl.BlockSpec((D, 96), lambda i: (0, 0)),
            pl.BlockSpec((1, 96), lambda i: (0, 0)),
            pl.BlockSpec((96, 48), lambda i: (0, 0)),
            pl.BlockSpec((1, 48), lambda i: (0, 0)),
        ],
        out_specs=pl.BlockSpec((NB, 48), lambda i: (i, 0)),
        out_shape=jax.ShapeDtypeStruct((N, 48), jnp.float32),
    )(t2, x, W3[:64], W3[64:], b3.reshape(1, 96), W4, b4.reshape(1, 48))


def kernel(x, edge_index, bn_gamma, bn_beta, W1, b1, cW1, cb1, W2, b2, cW2,
           cb2, W3, b3, W4, b4):
    pad = EPTP - EPT
    src = jnp.pad(edge_index[0].reshape(NS, EPT), ((0, 0), (0, pad)),
                  constant_values=0).reshape(NS, NCHP, CE)
    dst = jnp.pad(edge_index[1].reshape(NS, EPT), ((0, 0), (0, pad)),
                  constant_values=N).reshape(NS, NCHP, CE)

    (dis,) = _deg(dst)
    stats = _stats(x)
    q1 = _prologue(x, stats, bn_gamma, bn_beta, W1, b1, cW1, cb1)
    t1, _s1 = _tag1(q1, src, dst, dis)
    q2 = _mid(t1, W2, b2, cW2, cb2)
    t2, _s2 = _tag2(q2, src, dst, dis)
    return _epilogue(t2, x, W3, b3, W4, b4)


# speedup vs baseline: 1.1067x; 1.0129x over previous
import functools

import jax
import jax.numpy as jnp
from jax import lax
from jax.experimental import pallas as pl
from jax.experimental.pallas import tpu as pltpu
from jax.experimental.pallas import tpu_sc as plsc

N = 10000
E = 320000
D = 128

NC = 2
NS = 16
L = 16

EPT = E // NS
CE = 128
NCHP = (EPT + CE - 1) // CE
EPTP = NCHP * CE
ROW0_STEP = 640
RCH = 80
NPAD = 10240


def _gelu(x):
    return 0.5 * x * (1.0 + lax.erf(x * 0.7071067811865476))


def _rsqrt_newton(v):
    i = plsc.bitcast(v, jnp.int32)
    i = 0x5F3759DF - lax.shift_right_logical(i, 1)
    y = plsc.bitcast(i, jnp.float32)
    for _ in range(4):
        y = y * (1.5 - 0.5 * v * y * y)
    return y


def _make_deg_kernel():
    mesh = plsc.VectorSubcoreMesh(core_axis_name="c", subcore_axis_name="s")

    scratch = dict(
        dstall=pltpu.VMEM((NCHP, CE), jnp.int32),
        hist=pltpu.VMEM((N + L,), jnp.float32),
        degbuf=pltpu.VMEM((640,), jnp.float32),
        tmp640=pltpu.VMEM((640,), jnp.float32),
        disbuf=pltpu.VMEM((640,), jnp.float32),
        hists_sp=pltpu.VMEM_SHARED((NS, NPAD), jnp.float32),
    )

    def body(dst_hbm, dis_hbm, dstall, hist, degbuf, tmp640, disbuf,
             hists_sp):
        c = lax.axis_index("c")
        s = lax.axis_index("s")
        row0 = s * ROW0_STEP
        zf = jnp.zeros((L,), jnp.float32)
        pltpu.sync_copy(dst_hbm.at[s], dstall)

        def zhist(i, _):
            hist[pl.ds(i * L, L)] = zf
            return 0
        lax.fori_loop(0, (N + L) // L, zhist, 0)
        ones = jnp.full((L,), 1.0, jnp.float32)

        def dchunk_body(j, _):
            for u in range(CE // L):
                idx = dstall[j, pl.ds(u * L, L)]
                plsc.addupdate_scatter(hist, [idx], ones)
            return 0
        lax.fori_loop(0, NCHP, dchunk_body, 0)
        pltpu.sync_copy(hist.at[pl.ds(0, N)], hists_sp.at[s, pl.ds(0, N)])
        def zpad(g, _):
            hist[pl.ds(g * L, L)] = zf
            return 0
        lax.fori_loop(0, (NPAD - N) // L, zpad, 0)
        pltpu.sync_copy(hist.at[pl.ds(0, NPAD - N)],
                        hists_sp.at[s, pl.ds(N, NPAD - N)])
        plsc.subcore_barrier()

        def zdeg(g, _):
            degbuf[pl.ds(g * L, L)] = zf
            return 0
        lax.fori_loop(0, 40, zdeg, 0)
        for t in range(NS):
            pltpu.sync_copy(hists_sp.at[t, pl.ds(row0, 640)], tmp640)
            def addt(g, _):
                degbuf[pl.ds(g * L, L)] = (degbuf[pl.ds(g * L, L)]
                                           + tmp640[pl.ds(g * L, L)])
                return 0
            lax.fori_loop(0, 40, addt, 0)

        def mkdis(g, _):
            v = degbuf[pl.ds(g * L, L)]
            y = _rsqrt_newton(v)
            disbuf[pl.ds(g * L, L)] = jnp.where(v > 0.5, y, 0.0)
            return 0
        lax.fori_loop(0, 40, mkdis, 0)

        @pl.when(c == 0)
        def _():
            pltpu.sync_copy(disbuf, dis_hbm.at[pl.ds(row0, 640)])

    return pl.kernel(
        body,
        out_type=[jax.ShapeDtypeStruct((NPAD,), jnp.float32)],
        mesh=mesh,
        scratch_types=list(scratch.values()),
        compiler_params=pltpu.CompilerParams(
            use_tc_tiling_on_sc=False, needs_layout_passes=False),
    )


def _make_tag_kernel(F, npass):
    Fc = F // NC
    Fcb = Fc // npass
    UF = Fcb // L

    mesh = plsc.VectorSubcoreMesh(core_axis_name="c", subcore_axis_name="s")

    out_type = [
        jax.ShapeDtypeStruct((N, F), jnp.float32),
        jax.ShapeDtypeStruct((NC * npass * N, Fcb), jnp.float32),
    ]

    scratch = dict(
        bufA=pltpu.VMEM((CE, Fcb), jnp.float32),
        bufB=pltpu.VMEM((CE, Fcb), jnp.float32),
        dstall=pltpu.VMEM((NCHP, CE), jnp.int32),
        idxmod=pltpu.VMEM((NCHP, CE), jnp.int32),
        bufQ=pltpu.VMEM((RCH, Fcb), jnp.float32),
        bufAcc=pltpu.VMEM((RCH, Fcb), jnp.float32),
        bufQ2=pltpu.VMEM((RCH, Fcb), jnp.float32),
        bufAcc2=pltpu.VMEM((RCH, Fcb), jnp.float32),
        zero16=pltpu.VMEM((RCH, Fcb), jnp.float32),
        disbuf=pltpu.VMEM((640,), jnp.float32),
        acc_sp=pltpu.VMEM_SHARED((N + L, Fcb), jnp.float32),
        semA=pltpu.SemaphoreType.DMA,
        semB=pltpu.SemaphoreType.DMA,
    )

    def body(*refs):
        (q_hbm, src_hbm, dst_hbm, dis_in_hbm, out_hbm, s_hbm,
         bufA, bufB, dstall, idxmod, bufQ, bufAcc, bufQ2, bufAcc2,
         zero16, disbuf, acc_sp, semA, semB) = refs

        c = lax.axis_index("c")
        s = lax.axis_index("s")
        row0 = s * ROW0_STEP
        nrc = jnp.where(s == NS - 1, (N - 15 * ROW0_STEP) // RCH,
                        ROW0_STEP // RCH)
        zf = jnp.zeros((L,), jnp.float32)

        def zrow(r, _):
            for u in range(UF):
                zero16[r, pl.ds(u * L, L)] = zf
            return 0
        lax.fori_loop(0, RCH, zrow, 0)
        pltpu.sync_copy(dst_hbm.at[s], dstall)
        pltpu.sync_copy(dis_in_hbm.at[pl.ds(row0, 640)], disbuf)

        def crow_compute(r0_local, nrows, k, is_first, bq, ba):
            def rowfn(g, _):
                dvec = disbuf[pl.ds(r0_local + g * L, L)]
                for r16 in range(L):
                    r = g * L + r16
                    dv = dvec[r16]
                    for u in range(UF):
                        sl = pl.ds(u * L, L)
                        t = (bq[r, sl] if is_first
                             else bq[r, sl] + dv * ba[r, sl])
                        if k > 0:
                            t = dv * t
                        bq[r, sl] = t
                return 0
            lax.fori_loop(0, nrows // L, rowfn, 0)

        def crow_writeback(r0_local, nrows, k, p, bq_dma):
            r0 = row0 + r0_local
            colbase = c * Fc + p * Fcb
            blk = c * npass + p
            if k > 0:
                pltpu.sync_copy(bq_dma, s_hbm.at[pl.ds(blk * N + r0, nrows)])
            else:
                pltpu.sync_copy(
                    bq_dma, out_hbm.at[pl.ds(r0, nrows), pl.ds(colbase, Fcb)])

        def combine_phase(k, is_first, p):
            colbase = c * Fc + p * Fcb

            def ch_body(ch, _):
                r0 = row0 + ch * RCH
                pltpu.sync_copy(
                    q_hbm.at[k, pl.ds(r0, RCH), pl.ds(colbase, Fcb)], bufQ)
                if not is_first:
                    pltpu.sync_copy(acc_sp.at[pl.ds(r0, RCH)], bufAcc)
                    pltpu.sync_copy(zero16, acc_sp.at[pl.ds(r0, RCH)])
                crow_compute(ch * RCH, RCH, k, is_first, bufQ, bufAcc)
                crow_writeback(ch * RCH, RCH, k, p, bufQ)
                return 0
            lax.fori_loop(0, nrc, ch_body, 0)

        def gather_scatter_phase(p):
            cN = (c * npass + p) * N

            GB = (bufA, bufB)
            GS = (semA, semB)

            def fire_g(j, buf, sem):
                pltpu.async_copy(s_hbm.at[idxmod.at[j]], buf, sem)

            def drain(buf, sem):
                pltpu.make_async_copy(s_hbm.at[pl.ds(0, CE)], buf, sem).wait()

            def scat(j, buf):
                pltpu.sync_copy(buf, acc_sp.at[dstall.at[j]], add=True)

            pltpu.sync_copy(src_hbm.at[s], idxmod)

            def mkidx(j, _):
                for u in range(CE // L):
                    sl = pl.ds(u * L, L)
                    idxmod[j, sl] = idxmod[j, sl] + cN
                return 0
            lax.fori_loop(0, NCHP, mkidx, 0)

            fire_g(0, GB[0], GS[0])

            def pair(i, _):
                fire_g(2 * i + 1, GB[1], GS[1])
                drain(GB[0], GS[0])
                scat(2 * i, GB[0])
                fire_g(2 * i + 2, GB[0], GS[0])
                drain(GB[1], GS[1])
                scat(2 * i + 1, GB[1])
                return 0
            lax.fori_loop(0, NCHP // 2, pair, 0)
            drain(GB[0], GS[0])
            scat(NCHP - 1, GB[0])

        def init_zero_acc(ch, _):
            r0 = row0 + ch * RCH
            pltpu.sync_copy(zero16, acc_sp.at[pl.ds(r0, RCH)])
            return 0
        lax.fori_loop(0, nrc, init_zero_acc, 0)
        for p in range(npass):
            combine_phase(3, is_first=True, p=p)
        plsc.subcore_barrier()

        for k in (2, 1, 0):
            for p in range(npass):
                gather_scatter_phase(p)
                plsc.subcore_barrier()
                combine_phase(k, is_first=False, p=p)
                plsc.subcore_barrier()

    kfn = pl.kernel(
        body,
        out_type=out_type,
        mesh=mesh,
        scratch_types=list(scratch.values()),
        compiler_params=pltpu.CompilerParams(
            use_tc_tiling_on_sc=False, needs_layout_passes=False,
            internal_scratch_in_bytes=256 * 1024),
    )

    return kfn


_deg = _make_deg_kernel()
_tag1 = _make_tag_kernel(256, npass=2)
_tag2 = _make_tag_kernel(64, npass=1)


def _stats_body(x_ref, o_ref):
    xm = jnp.mean(x_ref[...], axis=0, keepdims=True)
    xv = jnp.mean(x_ref[...] * x_ref[...], axis=0, keepdims=True) - xm * xm
    o_ref[0:1, :] = xm
    o_ref[1:2, :] = xv


def _stats(x):
    return pl.pallas_call(
        _stats_body,
        out_shape=jax.ShapeDtypeStruct((2, D), jnp.float32),
    )(x)


NB = 1000


def _prologue_body(x_ref, st_ref, g_ref, be_ref, w1_ref, b1_ref, cw_ref,
                   cb_ref, q_ref, h1_ref):
    j = pl.program_id(1)

    @pl.when(j == 0)
    def _():
        mean = st_ref[0:1, :]
        var = st_ref[1:2, :]
        inv = g_ref[...] * lax.rsqrt(var + 1e-5)
        xn = (x_ref[...] - mean) * inv + be_ref[...]
        h1_ref[...] = _gelu(
            jnp.dot(xn, w1_ref[...], preferred_element_type=jnp.float32)
            + b1_ref[...])

    q = jnp.dot(h1_ref[...], cw_ref[0], preferred_element_type=jnp.float32)

    @pl.when(j == 0)
    def _():
        q_ref[0] = q + cb_ref[...]

    @pl.when(j > 0)
    def _():
        q_ref[0] = q


def _prologue(x, stats, gamma, beta, W1, b1, cW1, cb1):
    return pl.pallas_call(
        _prologue_body,
        grid=(N // NB, 4),
        in_specs=[
            pl.BlockSpec((NB, D), lambda i, j: (i, 0)),
            pl.BlockSpec((2, D), lambda i, j: (0, 0)),
            pl.BlockSpec((1, D), lambda i, j: (0, 0)),
            pl.BlockSpec((1, D), lambda i, j: (0, 0)),
            pl.BlockSpec((D, 512), lambda i, j: (0, 0)),
            pl.BlockSpec((1, 512), lambda i, j: (0, 0)),
            pl.BlockSpec((1, 512, 256), lambda i, j: (j, 0, 0)),
            pl.BlockSpec((1, 256), lambda i, j: (0, 0)),
        ],
        out_specs=pl.BlockSpec((1, NB, 256), lambda i, j: (j, i, 0)),
        out_shape=jax.ShapeDtypeStruct((4, N, 256), jnp.float32),
        scratch_shapes=[pltpu.VMEM((NB, 512), jnp.float32)],
    )(x, stats, gamma.reshape(1, D), beta.reshape(1, D), W1,
      b1.reshape(1, 512), cW1, cb1.reshape(1, 256))


def _mid_body(t_ref, w2_ref, b2_ref, cw_ref, cb_ref, q_ref, h2_ref):
    j = pl.program_id(1)

    @pl.when(j == 0)
    def _():
        a = _gelu(t_ref[...])
        h2_ref[...] = _gelu(
            jnp.dot(a, w2_ref[...], preferred_element_type=jnp.float32)
            + b2_ref[...])

    q = jnp.dot(h2_ref[...], cw_ref[0], preferred_element_type=jnp.float32)

    @pl.when(j == 0)
    def _():
        q_ref[0] = q + cb_ref[...]

    @pl.when(j > 0)
    def _():
        q_ref[0] = q


def _mid(t1, W2, b2, cW2, cb2):
    return pl.pallas_call(
        _mid_body,
        grid=(N // NB, 4),
        in_specs=[
            pl.BlockSpec((NB, 256), lambda i, j: (i, 0)),
            pl.BlockSpec((256, 128), lambda i, j: (0, 0)),
            pl.BlockSpec((1, 128), lambda i, j: (0, 0)),
            pl.BlockSpec((1, 128, 64), lambda i, j: (j, 0, 0)),
            pl.BlockSpec((1, 64), lambda i, j: (0, 0)),
        ],
        out_specs=pl.BlockSpec((1, NB, 64), lambda i, j: (j, i, 0)),
        out_shape=jax.ShapeDtypeStruct((4, N, 64), jnp.float32),
        scratch_shapes=[pltpu.VMEM((NB, 128), jnp.float32)],
    )(t1, W2, b2.reshape(1, 128), cW2, cb2.reshape(1, 64))


def _epi_body(t_ref, x_ref, w3a_ref, w3b_ref, b3_ref, w4_ref, b4_ref, o_ref):
    h3 = _gelu(t_ref[...])
    e = _gelu(
        jnp.dot(h3, w3a_ref[...], preferred_element_type=jnp.float32)
        + jnp.dot(x_ref[...], w3b_ref[...], preferred_element_type=jnp.float32)
        + b3_ref[...])
    o_ref[...] = (jnp.dot(e, w4_ref[...], preferred_element_type=jnp.float32)
                  + b4_ref[...])


def _epilogue(t2, x, W3, b3, W4, b4):
    return pl.pallas_call(
        _epi_body,
        grid=(N // NB,),
        in_specs=[
            pl.BlockSpec((NB, 64), lambda i: (i, 0)),
            pl.BlockSpec((NB, D), lambda i: (i, 0)),
            pl.BlockSpec((64, 96), lambda i: (0, 0)),
            pl.BlockSpec((D, 96), lambda i: (0, 0)),
            pl.BlockSpec((1, 96), lambda i: (0, 0)),
            pl.BlockSpec((96, 48), lambda i: (0, 0)),
            pl.BlockSpec((1, 48), lambda i: (0, 0)),
        ],
        out_specs=pl.BlockSpec((NB, 48), lambda i: (i, 0)),
        out_shape=jax.ShapeDtypeStruct((N, 48), jnp.float32),
    )(t2, x, W3[:64], W3[64:], b3.reshape(1, 96), W4, b4.reshape(1, 48))


def kernel(x, edge_index, bn_gamma, bn_beta, W1, b1, cW1, cb1, W2, b2, cW2,
           cb2, W3, b3, W4, b4):
    pad = EPTP - EPT
    src = jnp.pad(edge_index[0].reshape(NS, EPT), ((0, 0), (0, pad)),
                  constant_values=0).reshape(NS, NCHP, CE)
    dst = jnp.pad(edge_index[1].reshape(NS, EPT), ((0, 0), (0, pad)),
                  constant_values=N).reshape(NS, NCHP, CE)

    (dis,) = _deg(dst)
    stats = _stats(x)
    q1 = _prologue(x, stats, bn_gamma, bn_beta, W1, b1, cW1, cb1)
    t1, _s1 = _tag1(q1, src, dst, dis)
    q2 = _mid(t1, W2, b2, cW2, cb2)
    t2, _s2 = _tag2(q2, src, dst, dis)
    return _epilogue(t2, x, W3, b3, W4, b4)
